# baseline (speedup 1.0000x reference)
import functools

import jax
import jax.numpy as jnp
from jax import lax
from jax.experimental import pallas as pl
from jax.experimental.pallas import tpu as pltpu
from jax.experimental.pallas import tpu_sc as plsc

N = 100000
F = 128
C = 16
E = 3200000
NC = 2
NS = 16
NW = NC * NS
SB = 128
NSUB = 4
EB = SB * NSUB
NB = 198
EPAD = NW * NB * EB
NP = 100096
RPS = NP // NS

_mesh = plsc.VectorSubcoreMesh(core_axis_name="c", subcore_axis_name="s")
_sc_params = pltpu.CompilerParams(use_tc_tiling_on_sc=False)
_f32 = jnp.float32


def _sc_propagate(g, rows4, cols4, zeros):
    @functools.partial(
        pl.kernel,
        out_type=[jax.ShapeDtypeStruct((NP, C), _f32)] * 2,
        mesh=_mesh,
        scratch_types=[
            pltpu.VMEM((3, EB), jnp.int32),
            pltpu.VMEM((3, EB), jnp.int32),
            pltpu.VMEM((3, EB, C), _f32),
            pltpu.VMEM_SHARED((NP, C), _f32),
            pltpu.SemaphoreType.DMA,
            pltpu.SemaphoreType.DMA,
            pltpu.SemaphoreType.DMA,
            pltpu.SemaphoreType.DMA,
        ],
        compiler_params=_sc_params,
    )
    def k(g_hbm, row_hbm, col_hbm, z_hbm, oa_hbm, ob_hbm, ridx, cidx, buf,
          acc, irs, ics, gs, ss):
        cid = lax.axis_index("c")
        sid = lax.axis_index("s")
        wid = cid * NS + sid
        pltpu.sync_copy(z_hbm.at[pl.ds(sid * RPS, RPS)],
                        acc.at[pl.ds(sid * RPS, RPS)])
        plsc.subcore_barrier()

        def drain(sem):
            pltpu.make_async_copy(g_hbm.at[pl.ds(0, EB)], buf.at[0],
                                  sem).wait()

        def step(b, s):
            sp = (s + 2) % 3
            drain(gs)

            @pl.when(b >= 1)
            def _():
                drain(ss)

            @pl.when(b + 2 < NB)
            def _():
                pltpu.make_async_copy(row_hbm.at[wid, 0], ridx.at[0],
                                      irs).wait()
                pltpu.async_copy(g_hbm.at[ridx.at[sp]], buf.at[sp], gs)

            @pl.when(b + 3 < NB)
            def _():
                pltpu.async_copy(row_hbm.at[wid, b + 3], ridx.at[s], irs)

            @pl.when(b + 2 < NB)
            def _():
                pltpu.async_copy(col_hbm.at[wid, b + 2], cidx.at[sp], ics)

            @pl.when(b >= 2)
            def _():
                pltpu.make_async_copy(col_hbm.at[wid, 0], cidx.at[0],
                                      ics).wait()

            pltpu.async_copy(buf.at[s], acc.at[cidx.at[s]], ss, add=True)

        pltpu.sync_copy(row_hbm.at[wid, 0], ridx.at[0])
        pltpu.async_copy(g_hbm.at[ridx.at[0]], buf.at[0], gs)
        pltpu.sync_copy(row_hbm.at[wid, 1], ridx.at[1])
        pltpu.async_copy(g_hbm.at[ridx.at[1]], buf.at[1], gs)
        pltpu.sync_copy(col_hbm.at[wid, 0], cidx.at[0])
        pltpu.sync_copy(col_hbm.at[wid, 1], cidx.at[1])
        pltpu.async_copy(row_hbm.at[wid, 2], ridx.at[2], irs)

        @pl.loop(0, NB // 3)
        def _(t):
            step(3 * t, 0)
            step(3 * t + 1, 1)
            step(3 * t + 2, 2)

        drain(ss)
        plsc.subcore_barrier()
        sl = pl.ds(sid * RPS, RPS)

        @pl.when(cid == 0)
        def _():
            pltpu.sync_copy(acc.at[sl], oa_hbm.at[sl])

        @pl.when(cid == 1)
        def _():
            pltpu.sync_copy(acc.at[sl], ob_hbm.at[sl])

    return k(g, rows4, cols4, zeros)


def _sc_degree(rows4, ones, zeros1):
    @functools.partial(
        pl.kernel,
        out_type=[jax.ShapeDtypeStruct((NP, C), _f32)] * 2,
        mesh=_mesh,
        scratch_types=[
            pltpu.VMEM((2, EB), jnp.int32),
            pltpu.VMEM((EB,), _f32),
            pltpu.VMEM((RPS,), _f32),
            pltpu.VMEM((RPS, C), _f32),
            pltpu.VMEM_SHARED((NP,), _f32),
            pltpu.SemaphoreType.DMA,
            pltpu.SemaphoreType.DMA,
        ],
        compiler_params=_sc_params,
    )
    def k(row_hbm, ones_hbm, z_hbm, oa_hbm, ob_hbm, ridx, onesb, dbuf,
          rbuf, acc, irs, ss):
        cid = lax.axis_index("c")
        sid = lax.axis_index("s")
        wid = cid * NS + sid
        pltpu.sync_copy(ones_hbm, onesb)
        pltpu.sync_copy(z_hbm.at[pl.ds(sid * RPS, RPS)],
                        acc.at[pl.ds(sid * RPS, RPS)])
        plsc.subcore_barrier()

        def scat(p):
            pltpu.async_copy(onesb, acc.at[ridx.at[p]], ss, add=True)

        def drain_ss():
            pltpu.make_async_copy(ones_hbm, onesb, ss).wait()

        pltpu.sync_copy(row_hbm.at[wid, 0], ridx.at[0])

        def stepd(b, p):
            q = 1 - p

            @pl.when(b >= 1)
            def _():
                pltpu.make_async_copy(row_hbm.at[wid, 0], ridx.at[0],
                                      irs).wait()

            scat(p)

            @pl.when(b >= 1)
            def _():
                drain_ss()

            @pl.when(b + 1 < NB)
            def _():
                pltpu.async_copy(row_hbm.at[wid, b + 1], ridx.at[q], irs)

        @pl.loop(0, NB // 2)
        def _(t):
            stepd(2 * t, 0)
            stepd(2 * t + 1, 1)

        drain_ss()
        plsc.subcore_barrier()
        sl = pl.ds(sid * RPS, RPS)
        pltpu.sync_copy(acc.at[sl], dbuf)

        @pl.loop(0, RPS // 16)
        def _(i):
            v = dbuf[pl.ds(i * 16, 16)]
            for kk in range(16):
                rbuf[i * 16 + kk, :] = jnp.broadcast_to(v[kk], (C,))

        @pl.when(cid == 0)
        def _():
            pltpu.sync_copy(rbuf, oa_hbm.at[sl])

        @pl.when(cid == 1)
        def _():
            pltpu.sync_copy(rbuf, ob_hbm.at[sl])

    return k(rows4, ones, zeros1)


def _tc_matmul(x, w):
    BM = 2000

    def body(x_ref, w_ref, o_ref):
        o_ref[...] = jnp.dot(x_ref[...], w_ref[...],
                             preferred_element_type=_f32)

    return pl.pallas_call(
        body,
        grid=(N // BM,),
        in_specs=[
            pl.BlockSpec((BM, F), lambda i: (i, 0)),
            pl.BlockSpec((F, C), lambda i: (0, 0)),
        ],
        out_specs=pl.BlockSpec((BM, C), lambda i: (i, 0)),
        out_shape=jax.ShapeDtypeStruct((NP, C), _f32),
    )(x, w)


_RF = NP * C // 128


def _flat(a):
    return a.reshape(_RF, 128)


def _tc_prep(dega, degb, h0):
    def body(da, db, h, g0_o, di_o, di2_o):
        deg = da[...] + db[...]
        pos = deg > 0.0
        di = jnp.where(pos, lax.rsqrt(deg), 0.0)
        di_o[...] = di
        di2_o[...] = jnp.where(pos, 1.0 / deg, 0.0)
        g0_o[...] = di * h[...]

    return pl.pallas_call(
        body,
        out_shape=[jax.ShapeDtypeStruct((_RF, 128), _f32)] * 3,
    )(_flat(dega), _flat(degb), _flat(h0))


def _tc_combine(sa, sb, scale):
    def body(a, b, s, o):
        o[...] = s[...] * (a[...] + b[...])

    return pl.pallas_call(
        body,
        out_shape=jax.ShapeDtypeStruct((_RF, 128), _f32),
    )(_flat(sa), _flat(sb), scale)


def kernel(x, edge_index, weight):
    row = edge_index[0].astype(jnp.int32)
    col = edge_index[1].astype(jnp.int32)
    padv = jnp.full((EPAD - E,), N, jnp.int32)
    rows4 = jnp.concatenate([row, padv]).reshape(NW, NB, EB)
    cols4 = jnp.concatenate([col, padv]).reshape(NW, NB, EB)
    zeros = jnp.zeros((NP, C), _f32)
    zeros1 = jnp.zeros((NP,), _f32)
    ones = jnp.ones((EB,), _f32)

    dega, degb = _sc_degree(rows4, ones, zeros1)
    h0 = _tc_matmul(x, weight)
    g0f, dinv, dinv2 = _tc_prep(dega, degb, h0)

    s1a, s1b = _sc_propagate(g0f.reshape(NP, C), rows4, cols4, zeros)
    g1f = _tc_combine(s1a, s1b, dinv2)

    s2a, s2b = _sc_propagate(g1f.reshape(NP, C), rows4, cols4, zeros)
    h2f = _tc_combine(s2a, s2b, dinv)
    return h2f.reshape(NP, C)[:N]


# speedup vs baseline: 1.2445x; 1.2445x over previous
import functools

import jax
import jax.numpy as jnp
from jax import lax
from jax.experimental import pallas as pl
from jax.experimental.pallas import tpu as pltpu
from jax.experimental.pallas import tpu_sc as plsc

N = 100000
F = 128
C = 16
E = 3200000
NC = 2
NS = 16
NW = NC * NS
SB = 128
NSUB = 4
EB = SB * NSUB
NB = 196
EPAD = NW * NB * EB
NP = 100096
RPS = NP // NS

_mesh = plsc.VectorSubcoreMesh(core_axis_name="c", subcore_axis_name="s")
_sc_params = pltpu.CompilerParams(use_tc_tiling_on_sc=False)
_f32 = jnp.float32


def _sc_propagate(g, rows4, cols4, zeros):
    @functools.partial(
        pl.kernel,
        out_type=[jax.ShapeDtypeStruct((NP, C), _f32)] * 2,
        mesh=_mesh,
        scratch_types=[
            pltpu.VMEM((2, EB), jnp.int32),
            pltpu.VMEM((2, EB), jnp.int32),
            pltpu.VMEM((2, EB, C), _f32),
            pltpu.VMEM_SHARED((NP, C), _f32),
            pltpu.SemaphoreType.DMA,
            pltpu.SemaphoreType.DMA,
            pltpu.SemaphoreType.DMA,
            pltpu.SemaphoreType.DMA,
        ],
        compiler_params=_sc_params,
    )
    def k(g_hbm, row_hbm, col_hbm, z_hbm, oa_hbm, ob_hbm, ridx, cidx, buf,
          acc, irs, ics, gs, ss):
        cid = lax.axis_index("c")
        sid = lax.axis_index("s")
        wid = cid * NS + sid
        pltpu.sync_copy(z_hbm.at[pl.ds(sid * RPS, RPS)],
                        acc.at[pl.ds(sid * RPS, RPS)])
        plsc.subcore_barrier()

        def drain(sem):
            pltpu.make_async_copy(g_hbm.at[pl.ds(0, EB)], buf.at[0],
                                  sem).wait()

        def step(b, p):
            q = 1 - p
            drain(gs)

            @pl.when(b + 2 < NB)
            def _():
                pltpu.async_copy(row_hbm.at[wid, b + 2], ridx.at[p], irs)

            @pl.when(b >= 1)
            def _():
                drain(ss)

            @pl.when(b + 1 < NB)
            def _():
                pltpu.async_copy(col_hbm.at[wid, b + 1], cidx.at[q], ics)
                pltpu.make_async_copy(row_hbm.at[wid, 0], ridx.at[0],
                                      irs).wait()
                pltpu.async_copy(g_hbm.at[ridx.at[q]], buf.at[q], gs)

            @pl.when(b >= 1)
            def _():
                pltpu.make_async_copy(col_hbm.at[wid, 0], cidx.at[0],
                                      ics).wait()

            pltpu.async_copy(buf.at[p], acc.at[cidx.at[p]], ss, add=True)

        pltpu.sync_copy(row_hbm.at[wid, 0], ridx.at[0])
        pltpu.sync_copy(col_hbm.at[wid, 0], cidx.at[0])
        pltpu.async_copy(g_hbm.at[ridx.at[0]], buf.at[0], gs)
        pltpu.async_copy(row_hbm.at[wid, 1], ridx.at[1], irs)

        @pl.loop(0, NB // 2)
        def _(t):
            step(2 * t, 0)
            step(2 * t + 1, 1)

        drain(ss)
        plsc.subcore_barrier()
        sl = pl.ds(sid * RPS, RPS)

        @pl.when(cid == 0)
        def _():
            pltpu.sync_copy(acc.at[sl], oa_hbm.at[sl])

        @pl.when(cid == 1)
        def _():
            pltpu.sync_copy(acc.at[sl], ob_hbm.at[sl])

    return k(g, rows4, cols4, zeros)


def _sc_degree(rows4, ones, zeros1):
    @functools.partial(
        pl.kernel,
        out_type=[jax.ShapeDtypeStruct((NP, C), _f32)] * 2,
        mesh=_mesh,
        scratch_types=[
            pltpu.VMEM((2, EB), jnp.int32),
            pltpu.VMEM((EB,), _f32),
            pltpu.VMEM((RPS,), _f32),
            pltpu.VMEM((RPS, C), _f32),
            pltpu.VMEM_SHARED((NP,), _f32),
            pltpu.SemaphoreType.DMA,
            pltpu.SemaphoreType.DMA,
        ],
        compiler_params=_sc_params,
    )
    def k(row_hbm, ones_hbm, z_hbm, oa_hbm, ob_hbm, ridx, onesb, dbuf,
          rbuf, acc, irs, ss):
        cid = lax.axis_index("c")
        sid = lax.axis_index("s")
        wid = cid * NS + sid
        pltpu.sync_copy(ones_hbm, onesb)
        pltpu.sync_copy(z_hbm.at[pl.ds(sid * RPS, RPS)],
                        acc.at[pl.ds(sid * RPS, RPS)])
        plsc.subcore_barrier()

        def scat(p):
            pltpu.async_copy(onesb, acc.at[ridx.at[p]], ss, add=True)

        def drain_ss():
            pltpu.make_async_copy(ones_hbm, onesb, ss).wait()

        pltpu.sync_copy(row_hbm.at[wid, 0], ridx.at[0])

        def stepd(b, p):
            q = 1 - p

            @pl.when(b >= 1)
            def _():
                pltpu.make_async_copy(row_hbm.at[wid, 0], ridx.at[0],
                                      irs).wait()

            scat(p)

            @pl.when(b >= 1)
            def _():
                drain_ss()

            @pl.when(b + 1 < NB)
            def _():
                pltpu.async_copy(row_hbm.at[wid, b + 1], ridx.at[q], irs)

        @pl.loop(0, NB // 2)
        def _(t):
            stepd(2 * t, 0)
            stepd(2 * t + 1, 1)

        drain_ss()
        plsc.subcore_barrier()
        sl = pl.ds(sid * RPS, RPS)
        pltpu.sync_copy(acc.at[sl], dbuf)

        @pl.loop(0, RPS // 16)
        def _(i):
            v = dbuf[pl.ds(i * 16, 16)]
            for kk in range(16):
                rbuf[i * 16 + kk, :] = jnp.broadcast_to(v[kk], (C,))

        @pl.when(cid == 0)
        def _():
            pltpu.sync_copy(rbuf, oa_hbm.at[sl])

        @pl.when(cid == 1)
        def _():
            pltpu.sync_copy(rbuf, ob_hbm.at[sl])

    return k(rows4, ones, zeros1)


def _tc_matmul(x, w):
    BM = 2000

    def body(x_ref, w_ref, o_ref):
        o_ref[...] = jnp.dot(x_ref[...], w_ref[...],
                             preferred_element_type=_f32)

    return pl.pallas_call(
        body,
        grid=(N // BM,),
        in_specs=[
            pl.BlockSpec((BM, F), lambda i: (i, 0)),
            pl.BlockSpec((F, C), lambda i: (0, 0)),
        ],
        out_specs=pl.BlockSpec((BM, C), lambda i: (i, 0)),
        out_shape=jax.ShapeDtypeStruct((NP, C), _f32),
    )(x, w)


_RF = NP * C // 128


def _flat(a):
    return a.reshape(_RF, 128)


def _tc_prep(dega, degb, h0):
    def body(da, db, h, g0_o, di_o, di2_o):
        deg = da[...] + db[...]
        pos = deg > 0.0
        di = jnp.where(pos, lax.rsqrt(deg), 0.0)
        di_o[...] = di
        di2_o[...] = jnp.where(pos, 1.0 / deg, 0.0)
        g0_o[...] = di * h[...]

    return pl.pallas_call(
        body,
        out_shape=[jax.ShapeDtypeStruct((_RF, 128), _f32)] * 3,
    )(_flat(dega), _flat(degb), _flat(h0))


def _tc_combine(sa, sb, scale):
    def body(a, b, s, o):
        o[...] = s[...] * (a[...] + b[...])

    return pl.pallas_call(
        body,
        out_shape=jax.ShapeDtypeStruct((_RF, 128), _f32),
    )(_flat(sa), _flat(sb), scale)


def kernel(x, edge_index, weight):
    row = edge_index[0].astype(jnp.int32)
    col = edge_index[1].astype(jnp.int32)
    padv = jnp.full((EPAD - E,), N, jnp.int32)
    rows4 = jnp.concatenate([row, padv]).reshape(NW, NB, EB)
    cols4 = jnp.concatenate([col, padv]).reshape(NW, NB, EB)
    zeros = jnp.zeros((NP, C), _f32)
    zeros1 = jnp.zeros((NP,), _f32)
    ones = jnp.ones((EB,), _f32)

    dega, degb = _sc_degree(rows4, ones, zeros1)
    h0 = _tc_matmul(x, weight)
    g0f, dinv, dinv2 = _tc_prep(dega, degb, h0)

    s1a, s1b = _sc_propagate(g0f.reshape(NP, C), rows4, cols4, zeros)
    g1f = _tc_combine(s1a, s1b, dinv2)

    s2a, s2b = _sc_propagate(g1f.reshape(NP, C), rows4, cols4, zeros)
    h2f = _tc_combine(s2a, s2b, dinv)
    return h2f.reshape(NP, C)[:N]


# speedup vs baseline: 1.2517x; 1.0058x over previous
import functools

import jax
import jax.numpy as jnp
from jax import lax
from jax.experimental import pallas as pl
from jax.experimental.pallas import tpu as pltpu
from jax.experimental.pallas import tpu_sc as plsc

N = 100000
F = 128
C = 16
E = 3200000
NC = 2
NS = 16
NW = NC * NS
SB = 128
NSUB = 4
EB = SB * NSUB
NB = 196
EPAD = NW * NB * EB
NP = 100096
RPS = NP // NS

_mesh = plsc.VectorSubcoreMesh(core_axis_name="c", subcore_axis_name="s")
_sc_params = pltpu.CompilerParams(use_tc_tiling_on_sc=False)
_f32 = jnp.float32


CH = C // 2
NBT = EPAD // EB
NBC = NBT // NS


def _sc_propagate(g, rows4, cols4, zeros):
    @functools.partial(
        pl.kernel,
        out_type=[jax.ShapeDtypeStruct((NP, C), _f32)] * 2,
        mesh=_mesh,
        scratch_types=[
            pltpu.VMEM((2, EB), jnp.int32),
            pltpu.VMEM((2, EB), jnp.int32),
            pltpu.VMEM((2, EB, CH), _f32),
            pltpu.VMEM_SHARED((NP, CH), _f32),
            pltpu.VMEM_SHARED((NP, CH), _f32),
            pltpu.SemaphoreType.DMA,
            pltpu.SemaphoreType.DMA,
            pltpu.SemaphoreType.DMA,
            pltpu.SemaphoreType.DMA,
        ],
        compiler_params=_sc_params,
    )
    def k(g_hbm, row_hbm, col_hbm, z_hbm, oa_hbm, ob_hbm, ridx, cidx, buf,
          gst, acc, irs, ics, gs, ss):
        cid = lax.axis_index("c")
        sid = lax.axis_index("s")
        sl = pl.ds(sid * RPS, RPS)
        pltpu.sync_copy(g_hbm.at[sl, pl.ds(cid * CH, CH)], gst.at[sl])
        pltpu.sync_copy(z_hbm.at[sl, pl.ds(0, CH)], acc.at[sl])
        plsc.subcore_barrier()

        def drain(sem):
            pltpu.make_async_copy(z_hbm.at[pl.ds(0, EB), pl.ds(0, CH)],
                                  buf.at[0], sem).wait()

        def step(b, p):
            q = 1 - p
            drain(gs)

            @pl.when(b + 2 < NBC)
            def _():
                pltpu.async_copy(row_hbm.at[sid * NBC + b + 2], ridx.at[p],
                                 irs)

            @pl.when(b >= 1)
            def _():
                drain(ss)

            @pl.when(b + 1 < NBC)
            def _():
                pltpu.async_copy(col_hbm.at[sid * NBC + b + 1], cidx.at[q],
                                 ics)
                pltpu.make_async_copy(row_hbm.at[0], ridx.at[0],
                                      irs).wait()
                pltpu.async_copy(gst.at[ridx.at[q]], buf.at[q], gs)

            @pl.when(b >= 1)
            def _():
                pltpu.make_async_copy(col_hbm.at[0], cidx.at[0],
                                      ics).wait()

            pltpu.async_copy(buf.at[p], acc.at[cidx.at[p]], ss, add=True)

        pltpu.sync_copy(row_hbm.at[sid * NBC], ridx.at[0])
        pltpu.sync_copy(col_hbm.at[sid * NBC], cidx.at[0])
        pltpu.async_copy(gst.at[ridx.at[0]], buf.at[0], gs)
        pltpu.async_copy(row_hbm.at[sid * NBC + 1], ridx.at[1], irs)

        @pl.loop(0, NBC // 2)
        def _(t):
            step(2 * t, 0)
            step(2 * t + 1, 1)

        drain(ss)
        plsc.subcore_barrier()

        @pl.when(cid == 0)
        def _():
            pltpu.sync_copy(acc.at[sl], oa_hbm.at[sl, pl.ds(0, CH)])

        @pl.when(cid == 1)
        def _():
            pltpu.sync_copy(acc.at[sl], ob_hbm.at[sl, pl.ds(CH, CH)])

    return k(g, rows4, cols4, zeros)


def _sc_degree(rows4, ones, zeros1):
    @functools.partial(
        pl.kernel,
        out_type=[jax.ShapeDtypeStruct((NP, C), _f32)] * 2,
        mesh=_mesh,
        scratch_types=[
            pltpu.VMEM((2, EB), jnp.int32),
            pltpu.VMEM((EB,), _f32),
            pltpu.VMEM((RPS,), _f32),
            pltpu.VMEM((RPS, C), _f32),
            pltpu.VMEM_SHARED((NP,), _f32),
            pltpu.SemaphoreType.DMA,
            pltpu.SemaphoreType.DMA,
        ],
        compiler_params=_sc_params,
    )
    def k(row_hbm, ones_hbm, z_hbm, oa_hbm, ob_hbm, ridx, onesb, dbuf,
          rbuf, acc, irs, ss):
        cid = lax.axis_index("c")
        sid = lax.axis_index("s")
        wid = cid * NS + sid
        pltpu.sync_copy(ones_hbm, onesb)
        pltpu.sync_copy(z_hbm.at[pl.ds(sid * RPS, RPS)],
                        acc.at[pl.ds(sid * RPS, RPS)])
        plsc.subcore_barrier()

        def scat(p):
            pltpu.async_copy(onesb, acc.at[ridx.at[p]], ss, add=True)

        def drain_ss():
            pltpu.make_async_copy(ones_hbm, onesb, ss).wait()

        nbd = NBT // NW
        pltpu.sync_copy(row_hbm.at[wid * nbd], ridx.at[0])

        def stepd(b, p):
            q = 1 - p

            @pl.when(b >= 1)
            def _():
                pltpu.make_async_copy(row_hbm.at[0], ridx.at[0],
                                      irs).wait()

            scat(p)

            @pl.when(b >= 1)
            def _():
                drain_ss()

            @pl.when(b + 1 < nbd)
            def _():
                pltpu.async_copy(row_hbm.at[wid * nbd + b + 1], ridx.at[q],
                                 irs)

        @pl.loop(0, nbd // 2)
        def _(t):
            stepd(2 * t, 0)
            stepd(2 * t + 1, 1)

        drain_ss()
        plsc.subcore_barrier()
        sl = pl.ds(sid * RPS, RPS)
        pltpu.sync_copy(acc.at[sl], dbuf)

        @pl.loop(0, RPS // 16)
        def _(i):
            v = dbuf[pl.ds(i * 16, 16)]
            for kk in range(16):
                rbuf[i * 16 + kk, :] = jnp.broadcast_to(v[kk], (C,))

        @pl.when(cid == 0)
        def _():
            pltpu.sync_copy(rbuf, oa_hbm.at[sl])

        @pl.when(cid == 1)
        def _():
            pltpu.sync_copy(rbuf, ob_hbm.at[sl])

    return k(rows4, ones, zeros1)


def _tc_matmul(x, w):
    BM = 2000

    def body(x_ref, w_ref, o_ref):
        o_ref[...] = jnp.dot(x_ref[...], w_ref[...],
                             preferred_element_type=_f32)

    return pl.pallas_call(
        body,
        grid=(N // BM,),
        in_specs=[
            pl.BlockSpec((BM, F), lambda i: (i, 0)),
            pl.BlockSpec((F, C), lambda i: (0, 0)),
        ],
        out_specs=pl.BlockSpec((BM, C), lambda i: (i, 0)),
        out_shape=jax.ShapeDtypeStruct((NP, C), _f32),
    )(x, w)


_RF = NP * C // 128


def _flat(a):
    return a.reshape(_RF, 128)


def _tc_prep(dega, degb, h0):
    def body(da, db, h, g0_o, di_o, di2_o):
        deg = da[...] + db[...]
        pos = deg > 0.0
        di = jnp.where(pos, lax.rsqrt(deg), 0.0)
        di_o[...] = di
        di2_o[...] = jnp.where(pos, 1.0 / deg, 0.0)
        g0_o[...] = di * h[...]

    return pl.pallas_call(
        body,
        out_shape=[jax.ShapeDtypeStruct((_RF, 128), _f32)] * 3,
    )(_flat(dega), _flat(degb), _flat(h0))


def _tc_combine(sa, sb, scale):
    def body(a, b, s, o):
        lane = lax.broadcasted_iota(jnp.int32, (_RF, 128), 1)
        sel = lane % C < CH
        o[...] = s[...] * jnp.where(sel, a[...], b[...])

    return pl.pallas_call(
        body,
        out_shape=jax.ShapeDtypeStruct((_RF, 128), _f32),
    )(_flat(sa), _flat(sb), scale)


def kernel(x, edge_index, weight):
    row = edge_index[0].astype(jnp.int32)
    col = edge_index[1].astype(jnp.int32)
    padv = jnp.full((EPAD - E,), N, jnp.int32)
    rows4 = jnp.concatenate([row, padv]).reshape(NBT, EB)
    cols4 = jnp.concatenate([col, padv]).reshape(NBT, EB)
    zeros = jnp.zeros((NP, C), _f32)
    zeros1 = jnp.zeros((NP,), _f32)
    ones = jnp.ones((EB,), _f32)

    dega, degb = _sc_degree(rows4, ones, zeros1)
    h0 = _tc_matmul(x, weight)
    g0f, dinv, dinv2 = _tc_prep(dega, degb, h0)

    s1a, s1b = _sc_propagate(g0f.reshape(NP, C), rows4, cols4, zeros)
    g1f = _tc_combine(s1a, s1b, dinv2)

    s2a, s2b = _sc_propagate(g1f.reshape(NP, C), rows4, cols4, zeros)
    h2f = _tc_combine(s2a, s2b, dinv)
    return h2f.reshape(NP, C)[:N]
